# baseline (speedup 1.0000x reference)
import functools

import jax
import jax.numpy as jnp
from jax import lax
from jax.experimental import pallas as pl
from jax.experimental.pallas import tpu as pltpu
from jax.experimental.pallas import tpu_sc as plsc

NE = 8
HID = 1024
FF = 1408
T = 2048
NPAIR = 2 * T
BM = 128
NP_MOE = NPAIR + NE * BM
NP = NP_MOE + T
G = NP // BM
NEG = -1e30


def _meta_body(x_ref, rwp_ref, pairw_ref, pos_ref, gate_ref, te_ref,
               oh_ref, rk_ref):
    logits = jnp.dot(x_ref[...], rwp_ref[...],
                     preferred_element_type=jnp.float32)
    lane = lax.broadcasted_iota(jnp.int32, (T, 128), 1)
    l = jnp.where(lane < NE, logits, NEG)
    m0 = jnp.max(l, axis=1, keepdims=True)
    e0 = jnp.min(jnp.where(l >= m0, lane, 9999), axis=1, keepdims=True)
    l2 = jnp.where(lane == e0, NEG, l)
    m1 = jnp.max(l2, axis=1, keepdims=True)
    e1 = jnp.min(jnp.where(l2 >= m1, lane, 9999), axis=1, keepdims=True)
    w0 = 1.0 / (1.0 + jnp.exp(m1 - m0))
    pairw_ref[:T, :] = w0
    pairw_ref[T:, :] = 1.0 - w0
    g = logits[:, NE:NE + 1]
    gate_ref[...] = 1.0 / (1.0 + jnp.exp(-g))

    lane2 = lax.broadcasted_iota(jnp.int32, (T, 128), 1)
    oh_ref[:T, :] = (lane2 == e0).astype(jnp.float32)
    oh_ref[T:, :] = (lane2 == e1).astype(jnp.float32)

    ri = lax.broadcasted_iota(jnp.int32, (BM, BM), 0)
    ci = lax.broadcasted_iota(jnp.int32, (BM, BM), 1)
    stril = (ri > ci).astype(jnp.float32)
    striu = (ri < ci).astype(jnp.float32)
    carry = jnp.zeros((1, 128), jnp.float32)
    for b in range(NPAIR // BM):
        blk = oh_ref[b * BM:(b + 1) * BM, :]
        ex = jnp.dot(stril, blk, preferred_element_type=jnp.float32)
        rk_ref[b * BM:(b + 1) * BM, :] = ex + carry
        carry = carry + jnp.sum(blk, axis=0, keepdims=True)

    counts = carry[0:1, :]
    lr = lax.broadcasted_iota(jnp.int32, (1, 128), 1)
    ci32 = counts.astype(jnp.int32)
    cpad = jnp.where(lr < NE, ((ci32 + BM - 1) // BM) * BM, 0)
    cpadf = cpad.astype(jnp.float32)
    offs = jnp.dot(cpadf, striu, preferred_element_type=jnp.float32)
    oend = jnp.where(lr < NE, offs + cpadf, 1e9)

    for b in range(NPAIR // BM):
        sl = pl.ds(b * BM, BM)
        posb = jnp.sum((rk_ref[sl, :] + offs) * oh_ref[sl, :],
                       axis=1, keepdims=True)
        pos_ref[sl, :] = posb.astype(jnp.int32)

    gcol = lax.broadcasted_iota(jnp.int32, (64, 1), 0) * BM
    te = jnp.sum((gcol.astype(jnp.float32) >= oend).astype(jnp.float32),
                 axis=1, keepdims=True).astype(jnp.int32)
    te = jnp.where(gcol >= NP_MOE, NE, jnp.minimum(te, NE - 1))
    te_ref[...] = te


@jax.jit
def _meta(x, rwp_t):
    return pl.pallas_call(
        _meta_body,
        in_specs=[pl.BlockSpec((T, HID), lambda: (0, 0)),
                  pl.BlockSpec((HID, 128), lambda: (0, 0))],
        out_specs=[pl.BlockSpec((NPAIR, 1), lambda: (0, 0)),
                   pl.BlockSpec((NPAIR, 1), lambda: (0, 0)),
                   pl.BlockSpec((T, 1), lambda: (0, 0)),
                   pl.BlockSpec((64, 1), lambda: (0, 0))],
        out_shape=[jax.ShapeDtypeStruct((NPAIR, 1), jnp.float32),
                   jax.ShapeDtypeStruct((NPAIR, 1), jnp.int32),
                   jax.ShapeDtypeStruct((T, 1), jnp.float32),
                   jax.ShapeDtypeStruct((64, 1), jnp.int32)],
        scratch_shapes=[pltpu.VMEM((NPAIR, 128), jnp.float32),
                        pltpu.VMEM((NPAIR, 128), jnp.float32)],
    )(x, rwp_t)


_PW = NPAIR // 16
_RW = NP_MOE // 32
_SW = T // 32


def _disp_body(x_hbm, pos_hbm, pairw_hbm, gate_hbm, xs_hbm, roww_hbm,
               idx2, vtok, vw, idxg, rows, rbuf, zi, zf, tok_sh, w_sh, sem):
    c = lax.axis_index("c")
    s = lax.axis_index("s")
    w = c * 16 + s
    i16 = lax.iota(jnp.int32, 16)

    for j in range(20):
        zi[pl.ds(j * 16, 16)] = jnp.zeros((16,), jnp.int32)
        zf[pl.ds(j * 16, 16)] = jnp.zeros((16,), jnp.float32)
    pltpu.sync_copy(zi, tok_sh.at[pl.ds(s * 320, 320)])
    pltpu.sync_copy(zf, w_sh.at[pl.ds(s * 320, 320)])
    plsc.subcore_barrier()

    base = s * _PW
    for j in range(2):
        pltpu.sync_copy(pos_hbm.at[pl.ds(base + j * 128, 128)], idx2.at[j])
        pltpu.sync_copy(pairw_hbm.at[pl.ds(base + j * 128, 128)], vw.at[j])
        for k in range(8):
            vtok[j, pl.ds(k * 16, 16)] = (
                (base + j * 128 + k * 16 + i16) & (T - 1))
    for j in range(2):
        pltpu.sync_copy(vtok.at[j], tok_sh.at[idx2.at[j]], add=True)
        pltpu.sync_copy(vw.at[j], w_sh.at[idx2.at[j]], add=True)
    plsc.subcore_barrier()

    rbase = c * (NP_MOE // 2) + s * _RW
    for h in range(2):
        pltpu.sync_copy(tok_sh.at[pl.ds(rbase + h * 80, 80)], idxg.at[h])
        pltpu.async_copy(x_hbm.at[idxg.at[h]], rows, sem).wait()
        pltpu.sync_copy(rows, xs_hbm.at[pl.ds(rbase + h * 80, 80)])
    pltpu.sync_copy(w_sh.at[pl.ds(rbase, _RW)], rbuf)
    pltpu.sync_copy(rbuf, roww_hbm.at[pl.ds(rbase, _RW)])

    tbase = w * _SW
    pltpu.sync_copy(x_hbm.at[pl.ds(tbase, _SW)], rows.at[pl.ds(0, _SW)])
    pltpu.sync_copy(rows.at[pl.ds(0, _SW)],
                    xs_hbm.at[pl.ds(NP_MOE + tbase, _SW)])
    pltpu.sync_copy(gate_hbm.at[pl.ds(tbase, _SW)], rbuf.at[pl.ds(0, _SW)])
    pltpu.sync_copy(rbuf.at[pl.ds(0, _SW)],
                    roww_hbm.at[pl.ds(NP_MOE + tbase, _SW)])


@jax.jit
def _dispatch(x, pos, pairw, gate):
    mesh = plsc.VectorSubcoreMesh(core_axis_name="c", subcore_axis_name="s")
    f = functools.partial(
        pl.kernel,
        out_type=(jax.ShapeDtypeStruct((NP, HID), jnp.float32),
                  jax.ShapeDtypeStruct((NP,), jnp.float32)),
        mesh=mesh,
        scratch_types=[
            pltpu.VMEM((2, 128), jnp.int32),
            pltpu.VMEM((2, 128), jnp.int32),
            pltpu.VMEM((2, 128), jnp.float32),
            pltpu.VMEM((2, 80), jnp.int32),
            pltpu.VMEM((80, HID), jnp.float32),
            pltpu.VMEM((_RW,), jnp.float32),
            pltpu.VMEM((320,), jnp.int32),
            pltpu.VMEM((320,), jnp.float32),
            pltpu.VMEM_SHARED((NP_MOE,), jnp.int32),
            pltpu.VMEM_SHARED((NP_MOE,), jnp.float32),
            pltpu.SemaphoreType.DMA,
        ])(_disp_body)
    return f(x, pos, pairw, gate)


def _gmm_body(te_ref, xs_ref, gu_ref, dn_ref, roww_ref, ys_ref):
    xb = xs_ref[...].astype(jnp.bfloat16)
    gu = jnp.dot(xb, gu_ref[0], preferred_element_type=jnp.float32)
    h = (gu[:, :FF] * (1.0 / (1.0 + jnp.exp(-gu[:, :FF])))
         * gu[:, FF:]).astype(jnp.bfloat16)
    eo = jnp.dot(h, dn_ref[0], preferred_element_type=jnp.float32)
    ys_ref[...] = eo * roww_ref[...]


@jax.jit
def _gmm(te, xs, gu_t, dn_t, roww):
    grid_spec = pltpu.PrefetchScalarGridSpec(
        num_scalar_prefetch=1,
        grid=(G,),
        in_specs=[
            pl.BlockSpec((BM, HID), lambda g, te: (g, 0)),
            pl.BlockSpec((1, HID, 2 * FF), lambda g, te: (te[g], 0, 0)),
            pl.BlockSpec((1, FF, HID), lambda g, te: (te[g], 0, 0)),
            pl.BlockSpec((BM, 1), lambda g, te: (g, 0)),
        ],
        out_specs=pl.BlockSpec((BM, HID), lambda g, te: (g, 0)),
    )
    return pl.pallas_call(
        _gmm_body,
        grid_spec=grid_spec,
        out_shape=jax.ShapeDtypeStruct((NP, HID), jnp.float32),
        compiler_params=pltpu.CompilerParams(
            dimension_semantics=("arbitrary",)),
    )(te, xs, gu_t, dn_t, roww)


def _comb_body(ys_hbm, pos_hbm, out_hbm, idx4, abuf, bbuf, cbuf, sa, sb):
    c = lax.axis_index("c")
    s = lax.axis_index("s")
    w = c * 16 + s
    tbase = w * _SW
    pltpu.sync_copy(pos_hbm.at[pl.ds(tbase, 32)], idx4.at[0])
    pltpu.sync_copy(pos_hbm.at[pl.ds(tbase + 32, 32)], idx4.at[1])
    pltpu.sync_copy(pos_hbm.at[pl.ds(T + tbase, 32)], idx4.at[2])
    pltpu.sync_copy(pos_hbm.at[pl.ds(T + tbase + 32, 32)], idx4.at[3])
    for h in range(2):
        t0 = tbase + h * 32
        pltpu.async_copy(ys_hbm.at[idx4.at[h]], abuf, sa).wait()
        pltpu.async_copy(ys_hbm.at[idx4.at[2 + h]], bbuf, sb).wait()
        pltpu.sync_copy(ys_hbm.at[pl.ds(NP_MOE + t0, 32)], cbuf)

        def row(r, _):
            for q in range(HID // 16):
                sl = pl.ds(q * 16, 16)
                abuf[r, sl] = abuf[r, sl] + bbuf[r, sl] + cbuf[r, sl]
            return _
        lax.fori_loop(0, 32, row, 0)
        pltpu.sync_copy(abuf, out_hbm.at[pl.ds(t0, 32)])


@jax.jit
def _combine(ys, pos):
    mesh = plsc.VectorSubcoreMesh(core_axis_name="c", subcore_axis_name="s")
    f = functools.partial(
        pl.kernel,
        out_type=jax.ShapeDtypeStruct((T, HID), jnp.float32),
        mesh=mesh,
        scratch_types=[
            pltpu.VMEM((4, 32), jnp.int32),
            pltpu.VMEM((32, HID), jnp.float32),
            pltpu.VMEM((32, HID), jnp.float32),
            pltpu.VMEM((32, HID), jnp.float32),
            pltpu.SemaphoreType.DMA,
            pltpu.SemaphoreType.DMA,
        ])(_comb_body)
    return f(ys, pos)


def kernel(hidden_states, router_weight, gate_up_proj, down_proj,
           shared_gate_proj, shared_up_proj, shared_down_proj,
           shared_expert_gate_weight):
    B, S, H = hidden_states.shape
    x = hidden_states.reshape(-1, H)
    rwp = jnp.zeros((128, H), jnp.float32)
    rwp = rwp.at[:NE].set(router_weight)
    rwp = rwp.at[NE].set(shared_expert_gate_weight[0])
    sgu = jnp.concatenate([shared_gate_proj, shared_up_proj], axis=0)
    gu_t = jnp.concatenate(
        [gate_up_proj, sgu[None]], axis=0).transpose(0, 2, 1)
    dn_t = jnp.concatenate(
        [down_proj, shared_down_proj[None]], axis=0).transpose(0, 2, 1)
    gu_t = gu_t.astype(jnp.bfloat16)
    dn_t = dn_t.astype(jnp.bfloat16)

    pairw, pos, gate, te = _meta(x, rwp.T)
    pairw1 = pairw.reshape(NPAIR)
    pos1 = pos.reshape(NPAIR)
    gate1 = gate.reshape(T)
    te1 = te.reshape(64)[:G]

    xs, roww = _dispatch(x, pos1, pairw1, gate1)
    ys = _gmm(te1, xs, gu_t, dn_t, roww.reshape(NP, 1))
    out = _combine(ys, pos1)
    return out.reshape(B, S, H)


# speedup vs baseline: 1.3448x; 1.3448x over previous
import functools

import jax
import jax.numpy as jnp
from jax import lax
from jax.experimental import pallas as pl
from jax.experimental.pallas import tpu as pltpu
from jax.experimental.pallas import tpu_sc as plsc

NE = 8
HID = 1024
FF = 1408
T = 2048
NPAIR = 2 * T
BM = 128
NP = NPAIR + NE * BM
G = NP // BM
TB = 256
NEG = -1e30


def _meta_body(x_ref, rwp_ref, pairw_ref, pos_ref, gate_ref, te_ref,
               oh_ref, rk_ref):
    logits = lax.dot_general(
        x_ref[...], rwp_ref[...], (((1,), (1,)), ((), ())),
        preferred_element_type=jnp.float32)
    lane = lax.broadcasted_iota(jnp.int32, (T, 128), 1)
    l = jnp.where(lane < NE, logits, NEG)
    m0 = jnp.max(l, axis=1, keepdims=True)
    e0 = jnp.min(jnp.where(l >= m0, lane, 9999), axis=1, keepdims=True)
    l2 = jnp.where(lane == e0, NEG, l)
    m1 = jnp.max(l2, axis=1, keepdims=True)
    e1 = jnp.min(jnp.where(l2 >= m1, lane, 9999), axis=1, keepdims=True)
    w0 = 1.0 / (1.0 + jnp.exp(m1 - m0))
    pairw_ref[:T, :] = w0
    pairw_ref[T:, :] = 1.0 - w0
    g = logits[:, NE:NE + 1]
    gate_ref[...] = 1.0 / (1.0 + jnp.exp(-g))

    oh_ref[:T, :] = (lane == e0).astype(jnp.float32)
    oh_ref[T:, :] = (lane == e1).astype(jnp.float32)

    ri = lax.broadcasted_iota(jnp.int32, (BM, BM), 0)
    ci = lax.broadcasted_iota(jnp.int32, (BM, BM), 1)
    stril = (ri > ci).astype(jnp.float32)
    striu = (ri < ci).astype(jnp.float32)
    carry = jnp.zeros((1, 128), jnp.float32)
    for b in range(NPAIR // BM):
        blk = oh_ref[b * BM:(b + 1) * BM, :]
        ex = jnp.dot(stril, blk, preferred_element_type=jnp.float32)
        rk_ref[b * BM:(b + 1) * BM, :] = ex + carry
        carry = carry + jnp.sum(blk, axis=0, keepdims=True)

    counts = carry[0:1, :]
    lr = lax.broadcasted_iota(jnp.int32, (1, 128), 1)
    ci32 = counts.astype(jnp.int32)
    cpad = jnp.where(lr < NE, ((ci32 + BM - 1) // BM) * BM, 0)
    cpadf = cpad.astype(jnp.float32)
    offs = jnp.dot(cpadf, striu, preferred_element_type=jnp.float32)
    oend = jnp.where(lr < NE, offs + cpadf, 1e9)

    for b in range(NPAIR // BM):
        sl = pl.ds(b * BM, BM)
        posb = jnp.sum((rk_ref[sl, :] + offs) * oh_ref[sl, :],
                       axis=1, keepdims=True)
        pos_ref[sl, :] = posb.astype(jnp.int32)

    gcol = lax.broadcasted_iota(jnp.int32, (64, 1), 0) * BM
    te = jnp.sum((gcol.astype(jnp.float32) >= oend).astype(jnp.float32),
                 axis=1, keepdims=True).astype(jnp.int32)
    te_ref[...] = jnp.minimum(te, NE - 1)


@jax.jit
def _meta(x, rwp):
    return pl.pallas_call(
        _meta_body,
        in_specs=[pl.BlockSpec((T, HID), lambda: (0, 0)),
                  pl.BlockSpec((128, HID), lambda: (0, 0))],
        out_specs=[pl.BlockSpec((NPAIR, 1), lambda: (0, 0)),
                   pl.BlockSpec((NPAIR, 1), lambda: (0, 0)),
                   pl.BlockSpec((T, 1), lambda: (0, 0)),
                   pl.BlockSpec((64, 1), lambda: (0, 0))],
        out_shape=[jax.ShapeDtypeStruct((NPAIR, 1), jnp.float32),
                   jax.ShapeDtypeStruct((NPAIR, 1), jnp.int32),
                   jax.ShapeDtypeStruct((T, 1), jnp.float32),
                   jax.ShapeDtypeStruct((64, 1), jnp.int32)],
        scratch_shapes=[pltpu.VMEM((NPAIR, 128), jnp.float32),
                        pltpu.VMEM((NPAIR, 128), jnp.float32)],
    )(x, rwp)


_PW = NPAIR // 16
_RW = NP // 32


def _disp_body(x_hbm, pos_hbm, pairw_hbm, xs_hbm, roww_hbm,
               idx2, vtok, vw, idxg, rows, rbuf, zi, zf, tok_sh, w_sh, sem):
    c = lax.axis_index("c")
    s = lax.axis_index("s")
    i16 = lax.iota(jnp.int32, 16)

    for j in range(20):
        zi[pl.ds(j * 16, 16)] = jnp.zeros((16,), jnp.int32)
        zf[pl.ds(j * 16, 16)] = jnp.zeros((16,), jnp.float32)
    pltpu.sync_copy(zi, tok_sh.at[pl.ds(s * 320, 320)])
    pltpu.sync_copy(zf, w_sh.at[pl.ds(s * 320, 320)])
    plsc.subcore_barrier()

    base = s * _PW
    for j in range(2):
        pltpu.sync_copy(pos_hbm.at[pl.ds(base + j * 128, 128)], idx2.at[j])
        pltpu.sync_copy(pairw_hbm.at[pl.ds(base + j * 128, 128)], vw.at[j])
        for k in range(8):
            vtok[j, pl.ds(k * 16, 16)] = (
                (base + j * 128 + k * 16 + i16) & (T - 1))
    for j in range(2):
        pltpu.sync_copy(vtok.at[j], tok_sh.at[idx2.at[j]], add=True)
        pltpu.sync_copy(vw.at[j], w_sh.at[idx2.at[j]], add=True)
    plsc.subcore_barrier()

    rbase = c * (NP // 2) + s * _RW
    for h in range(2):
        pltpu.sync_copy(tok_sh.at[pl.ds(rbase + h * 80, 80)], idxg.at[h])
        pltpu.async_copy(x_hbm.at[idxg.at[h]], rows, sem).wait()
        pltpu.sync_copy(rows, xs_hbm.at[pl.ds(rbase + h * 80, 80)])
    pltpu.sync_copy(w_sh.at[pl.ds(rbase, _RW)], rbuf)
    pltpu.sync_copy(rbuf, roww_hbm.at[pl.ds(rbase, _RW)])


@jax.jit
def _dispatch(x, pos, pairw):
    mesh = plsc.VectorSubcoreMesh(core_axis_name="c", subcore_axis_name="s")
    f = functools.partial(
        pl.kernel,
        out_type=(jax.ShapeDtypeStruct((NP, HID), jnp.float32),
                  jax.ShapeDtypeStruct((NP,), jnp.float32)),
        mesh=mesh,
        scratch_types=[
            pltpu.VMEM((2, 128), jnp.int32),
            pltpu.VMEM((2, 128), jnp.int32),
            pltpu.VMEM((2, 128), jnp.float32),
            pltpu.VMEM((2, 80), jnp.int32),
            pltpu.VMEM((80, HID), jnp.float32),
            pltpu.VMEM((_RW,), jnp.float32),
            pltpu.VMEM((320,), jnp.int32),
            pltpu.VMEM((320,), jnp.float32),
            pltpu.VMEM_SHARED((NP,), jnp.int32),
            pltpu.VMEM_SHARED((NP,), jnp.float32),
            pltpu.SemaphoreType.DMA,
        ])(_disp_body)
    return f(x, pos, pairw)


def _gmm_body(te_ref, xs_ref, gu_ref, dn_ref, roww_ref, ys_ref,
              gub_ref, dnb_ref):
    g = pl.program_id(0)
    te = te_ref[g]
    prev = te_ref[jnp.maximum(g - 1, 0)]

    @pl.when((g == 0) | (te != prev))
    def _recast():
        gub_ref[...] = gu_ref[0].astype(jnp.bfloat16)
        dnb_ref[...] = dn_ref[0].astype(jnp.bfloat16)

    xb = xs_ref[...].astype(jnp.bfloat16)
    gu = lax.dot_general(xb, gub_ref[...], (((1,), (1,)), ((), ())),
                         preferred_element_type=jnp.float32)
    h = (gu[:, :FF] * (1.0 / (1.0 + jnp.exp(-gu[:, :FF])))
         * gu[:, FF:]).astype(jnp.bfloat16)
    eo = lax.dot_general(h, dnb_ref[...], (((1,), (1,)), ((), ())),
                         preferred_element_type=jnp.float32)
    ys_ref[...] = eo * roww_ref[...]


@jax.jit
def _gmm(te, xs, gup, dnp, roww):
    grid_spec = pltpu.PrefetchScalarGridSpec(
        num_scalar_prefetch=1,
        grid=(G,),
        in_specs=[
            pl.BlockSpec((BM, HID), lambda g, te: (g, 0)),
            pl.BlockSpec((1, 2 * FF, HID), lambda g, te: (te[g], 0, 0)),
            pl.BlockSpec((1, HID, FF), lambda g, te: (te[g], 0, 0)),
            pl.BlockSpec((BM, 1), lambda g, te: (g, 0)),
        ],
        out_specs=pl.BlockSpec((BM, HID), lambda g, te: (g, 0)),
        scratch_shapes=[pltpu.VMEM((2 * FF, HID), jnp.bfloat16),
                        pltpu.VMEM((HID, FF), jnp.bfloat16)],
    )
    return pl.pallas_call(
        _gmm_body,
        grid_spec=grid_spec,
        out_shape=jax.ShapeDtypeStruct((NP, HID), jnp.float32),
        compiler_params=pltpu.CompilerParams(
            dimension_semantics=("arbitrary",)),
    )(te, xs, gup, dnp, roww)


def _shared_body(x_ref, sg_ref, su_ref, sd_ref, gate_ref, out_ref,
                 sgb_ref, sub_ref, sdb_ref):
    @pl.when(pl.program_id(0) == 0)
    def _cast():
        sgb_ref[...] = sg_ref[...].astype(jnp.bfloat16)
        sub_ref[...] = su_ref[...].astype(jnp.bfloat16)
        sdb_ref[...] = sd_ref[...].astype(jnp.bfloat16)

    xb = x_ref[...].astype(jnp.bfloat16)
    gg = lax.dot_general(xb, sgb_ref[...], (((1,), (1,)), ((), ())),
                         preferred_element_type=jnp.float32)
    uu = lax.dot_general(xb, sub_ref[...], (((1,), (1,)), ((), ())),
                         preferred_element_type=jnp.float32)
    h = (gg * (1.0 / (1.0 + jnp.exp(-gg))) * uu).astype(jnp.bfloat16)
    eo = lax.dot_general(h, sdb_ref[...], (((1,), (1,)), ((), ())),
                         preferred_element_type=jnp.float32)
    out_ref[...] = eo * gate_ref[...]


@jax.jit
def _shared(x, sg, su, sd, gate):
    return pl.pallas_call(
        _shared_body,
        grid=(T // TB,),
        in_specs=[
            pl.BlockSpec((TB, HID), lambda g: (g, 0)),
            pl.BlockSpec((FF, HID), lambda g: (0, 0)),
            pl.BlockSpec((FF, HID), lambda g: (0, 0)),
            pl.BlockSpec((HID, FF), lambda g: (0, 0)),
            pl.BlockSpec((TB, 1), lambda g: (g, 0)),
        ],
        out_specs=pl.BlockSpec((TB, HID), lambda g: (g, 0)),
        out_shape=jax.ShapeDtypeStruct((T, HID), jnp.float32),
        scratch_shapes=[pltpu.VMEM((FF, HID), jnp.bfloat16),
                        pltpu.VMEM((FF, HID), jnp.bfloat16),
                        pltpu.VMEM((HID, FF), jnp.bfloat16)],
        compiler_params=pltpu.CompilerParams(
            dimension_semantics=("arbitrary",)),
    )(x, sg, su, sd, gate)


_SW = T // 32


def _comb_body(ys_hbm, sh_hbm, pos_hbm, out_hbm, idx4, abuf, bbuf, cbuf,
               sa, sb):
    c = lax.axis_index("c")
    s = lax.axis_index("s")
    w = c * 16 + s
    tbase = w * _SW
    pltpu.sync_copy(pos_hbm.at[pl.ds(tbase, 32)], idx4.at[0])
    pltpu.sync_copy(pos_hbm.at[pl.ds(tbase + 32, 32)], idx4.at[1])
    pltpu.sync_copy(pos_hbm.at[pl.ds(T + tbase, 32)], idx4.at[2])
    pltpu.sync_copy(pos_hbm.at[pl.ds(T + tbase + 32, 32)], idx4.at[3])
    for h in range(2):
        t0 = tbase + h * 32
        pltpu.async_copy(ys_hbm.at[idx4.at[h]], abuf, sa).wait()
        pltpu.async_copy(ys_hbm.at[idx4.at[2 + h]], bbuf, sb).wait()
        pltpu.sync_copy(sh_hbm.at[pl.ds(t0, 32)], cbuf)

        def row(r, carry):
            for q in range(HID // 16):
                sl = pl.ds(q * 16, 16)
                abuf[r, sl] = abuf[r, sl] + bbuf[r, sl] + cbuf[r, sl]
            return carry
        lax.fori_loop(0, 32, row, 0)
        pltpu.sync_copy(abuf, out_hbm.at[pl.ds(t0, 32)])


@jax.jit
def _combine(ys, sh, pos):
    mesh = plsc.VectorSubcoreMesh(core_axis_name="c", subcore_axis_name="s")
    f = functools.partial(
        pl.kernel,
        out_type=jax.ShapeDtypeStruct((T, HID), jnp.float32),
        mesh=mesh,
        scratch_types=[
            pltpu.VMEM((4, 32), jnp.int32),
            pltpu.VMEM((32, HID), jnp.float32),
            pltpu.VMEM((32, HID), jnp.float32),
            pltpu.VMEM((32, HID), jnp.float32),
            pltpu.SemaphoreType.DMA,
            pltpu.SemaphoreType.DMA,
        ])(_comb_body)
    return f(ys, sh, pos)


def kernel(hidden_states, router_weight, gate_up_proj, down_proj,
           shared_gate_proj, shared_up_proj, shared_down_proj,
           shared_expert_gate_weight):
    B, S, H = hidden_states.shape
    x = hidden_states.reshape(-1, H)
    rwp = jnp.zeros((128, H), jnp.float32)
    rwp = rwp.at[:NE].set(router_weight)
    rwp = rwp.at[NE].set(shared_expert_gate_weight[0])

    pairw, pos, gate, te = _meta(x, rwp)
    pairw1 = pairw.reshape(NPAIR)
    pos1 = pos.reshape(NPAIR)
    te1 = te.reshape(64)[:G]

    xs, roww = _dispatch(x, pos1, pairw1)
    ys = _gmm(te1, xs, gate_up_proj, down_proj, roww.reshape(NP, 1))
    sh = _shared(x, shared_gate_proj, shared_up_proj, shared_down_proj,
                 gate)
    out = _combine(ys, sh, pos1)
    return out.reshape(B, S, H)


# speedup vs baseline: 1.3765x; 1.0235x over previous
import functools

import jax
import jax.numpy as jnp
from jax import lax
from jax.experimental import pallas as pl
from jax.experimental.pallas import tpu as pltpu
from jax.experimental.pallas import tpu_sc as plsc

NE = 8
HID = 1024
FF = 1408
T = 2048
NPAIR = 2 * T
BM = 256
CB = 128
NP = NPAIR + NE * BM
G = NP // BM
TB = 256
NEG = -1e30


def _meta_body(x_ref, rwp_ref, pairw_ref, pos_ref, gate_ref, te_ref,
               oh_ref, rk_ref):
    logits = lax.dot_general(
        x_ref[...], rwp_ref[...], (((1,), (1,)), ((), ())),
        preferred_element_type=jnp.float32)
    lane = lax.broadcasted_iota(jnp.int32, (T, 128), 1)
    l = jnp.where(lane < NE, logits, NEG)
    m0 = jnp.max(l, axis=1, keepdims=True)
    e0 = jnp.min(jnp.where(l >= m0, lane, 9999), axis=1, keepdims=True)
    l2 = jnp.where(lane == e0, NEG, l)
    m1 = jnp.max(l2, axis=1, keepdims=True)
    e1 = jnp.min(jnp.where(l2 >= m1, lane, 9999), axis=1, keepdims=True)
    w0 = 1.0 / (1.0 + jnp.exp(m1 - m0))
    pairw_ref[:T, :] = w0
    pairw_ref[T:, :] = 1.0 - w0
    g = logits[:, NE:NE + 1]
    gate_ref[...] = 1.0 / (1.0 + jnp.exp(-g))

    oh_ref[:T, :] = (lane == e0).astype(jnp.float32)
    oh_ref[T:, :] = (lane == e1).astype(jnp.float32)

    ri = lax.broadcasted_iota(jnp.int32, (CB, CB), 0)
    ci = lax.broadcasted_iota(jnp.int32, (CB, CB), 1)
    stril = (ri > ci).astype(jnp.float32)
    striu = (ri < ci).astype(jnp.float32)
    carry = jnp.zeros((1, 128), jnp.float32)
    for b in range(NPAIR // CB):
        blk = oh_ref[b * CB:(b + 1) * CB, :]
        ex = jnp.dot(stril, blk, preferred_element_type=jnp.float32)
        rk_ref[b * CB:(b + 1) * CB, :] = ex + carry
        carry = carry + jnp.sum(blk, axis=0, keepdims=True)

    counts = carry[0:1, :]
    lr = lax.broadcasted_iota(jnp.int32, (1, 128), 1)
    ci32 = counts.astype(jnp.int32)
    cpad = jnp.where(lr < NE, ((ci32 + BM - 1) // BM) * BM, 0)
    cpadf = cpad.astype(jnp.float32)
    offs = jnp.dot(cpadf, striu, preferred_element_type=jnp.float32)
    oend = jnp.where(lr < NE, offs + cpadf, 1e9)

    for b in range(NPAIR // CB):
        sl = pl.ds(b * CB, CB)
        posb = jnp.sum((rk_ref[sl, :] + offs) * oh_ref[sl, :],
                       axis=1, keepdims=True)
        pos_ref[sl, :] = posb.astype(jnp.int32)

    gcol = lax.broadcasted_iota(jnp.int32, (32, 1), 0) * BM
    te = jnp.sum((gcol.astype(jnp.float32) >= oend).astype(jnp.float32),
                 axis=1, keepdims=True).astype(jnp.int32)
    te_ref[...] = jnp.minimum(te, NE - 1)


@jax.jit
def _meta(x, rwp):
    return pl.pallas_call(
        _meta_body,
        in_specs=[pl.BlockSpec((T, HID), lambda: (0, 0)),
                  pl.BlockSpec((128, HID), lambda: (0, 0))],
        out_specs=[pl.BlockSpec((NPAIR, 1), lambda: (0, 0)),
                   pl.BlockSpec((NPAIR, 1), lambda: (0, 0)),
                   pl.BlockSpec((T, 1), lambda: (0, 0)),
                   pl.BlockSpec((32, 1), lambda: (0, 0))],
        out_shape=[jax.ShapeDtypeStruct((NPAIR, 1), jnp.float32),
                   jax.ShapeDtypeStruct((NPAIR, 1), jnp.int32),
                   jax.ShapeDtypeStruct((T, 1), jnp.float32),
                   jax.ShapeDtypeStruct((32, 1), jnp.int32)],
        scratch_shapes=[pltpu.VMEM((NPAIR, 128), jnp.float32),
                        pltpu.VMEM((NPAIR, 128), jnp.float32)],
    )(x, rwp)


_PW = NPAIR // 16
_RW = NP // 32
_GC = 48
_ZW = NP // 16


def _disp_body(x_hbm, pos_hbm, pairw_hbm, xs_hbm, roww_hbm,
               idx2, vtok, vw, idxf, rows0, rows1, rbuf, zi, zf,
               tok_sh, w_sh, sem0, sem1):
    c = lax.axis_index("c")
    s = lax.axis_index("s")
    i16 = lax.iota(jnp.int32, 16)

    for j in range(_ZW // 16):
        zi[pl.ds(j * 16, 16)] = jnp.zeros((16,), jnp.int32)
        zf[pl.ds(j * 16, 16)] = jnp.zeros((16,), jnp.float32)
    pltpu.sync_copy(zi, tok_sh.at[pl.ds(s * _ZW, _ZW)])
    pltpu.sync_copy(zf, w_sh.at[pl.ds(s * _ZW, _ZW)])
    plsc.subcore_barrier()

    base = s * _PW
    for j in range(2):
        pltpu.sync_copy(pos_hbm.at[pl.ds(base + j * 128, 128)], idx2.at[j])
        pltpu.sync_copy(pairw_hbm.at[pl.ds(base + j * 128, 128)], vw.at[j])
        for k in range(8):
            vtok[j, pl.ds(k * 16, 16)] = (
                (base + j * 128 + k * 16 + i16) & (T - 1))
    for j in range(2):
        pltpu.sync_copy(vtok.at[j], tok_sh.at[idx2.at[j]], add=True)
        pltpu.sync_copy(vw.at[j], w_sh.at[idx2.at[j]], add=True)
    plsc.subcore_barrier()

    rbase = c * (NP // 2) + s * _RW
    pltpu.sync_copy(tok_sh.at[pl.ds(rbase, _RW)], idxf)
    bufs = (rows0, rows1)
    sems = (sem0, sem1)
    cps = []
    for h in range(2):
        cps.append(pltpu.async_copy(
            x_hbm.at[idxf.at[pl.ds(h * _GC, _GC)]], bufs[h], sems[h]))
    for h in range(4):
        cps[h].wait()
        pltpu.sync_copy(bufs[h % 2], xs_hbm.at[pl.ds(rbase + h * _GC, _GC)])
        if h + 2 < 4:
            cps.append(pltpu.async_copy(
                x_hbm.at[idxf.at[pl.ds((h + 2) * _GC, _GC)]],
                bufs[h % 2], sems[h % 2]))
    pltpu.sync_copy(w_sh.at[pl.ds(rbase, _RW)], rbuf)
    pltpu.sync_copy(rbuf, roww_hbm.at[pl.ds(rbase, _RW)])


@jax.jit
def _dispatch(x, pos, pairw):
    mesh = plsc.VectorSubcoreMesh(core_axis_name="c", subcore_axis_name="s")
    f = functools.partial(
        pl.kernel,
        out_type=(jax.ShapeDtypeStruct((NP, HID), jnp.float32),
                  jax.ShapeDtypeStruct((NP,), jnp.float32)),
        mesh=mesh,
        scratch_types=[
            pltpu.VMEM((2, 128), jnp.int32),
            pltpu.VMEM((2, 128), jnp.int32),
            pltpu.VMEM((2, 128), jnp.float32),
            pltpu.VMEM((_RW,), jnp.int32),
            pltpu.VMEM((_GC, HID), jnp.float32),
            pltpu.VMEM((_GC, HID), jnp.float32),
            pltpu.VMEM((_RW,), jnp.float32),
            pltpu.VMEM((_ZW,), jnp.int32),
            pltpu.VMEM((_ZW,), jnp.float32),
            pltpu.VMEM_SHARED((NP,), jnp.int32),
            pltpu.VMEM_SHARED((NP,), jnp.float32),
            pltpu.SemaphoreType.DMA,
            pltpu.SemaphoreType.DMA,
        ])(_disp_body)
    return f(x, pos, pairw)


def _gmm_body(te_ref, xs_ref, gu_ref, dn_ref, roww_ref, ys_ref,
              gub_ref, dnb_ref):
    g = pl.program_id(0)
    te = te_ref[g]
    prev = te_ref[jnp.maximum(g - 1, 0)]

    @pl.when((g == 0) | (te != prev))
    def _recast():
        gub_ref[...] = gu_ref[0].astype(jnp.bfloat16)
        dnb_ref[...] = dn_ref[0].astype(jnp.bfloat16)

    xb = xs_ref[...].astype(jnp.bfloat16)
    gu = lax.dot_general(xb, gub_ref[...], (((1,), (1,)), ((), ())),
                         preferred_element_type=jnp.float32)
    h = (gu[:, :FF] * (1.0 / (1.0 + jnp.exp(-gu[:, :FF])))
         * gu[:, FF:]).astype(jnp.bfloat16)
    eo = lax.dot_general(h, dnb_ref[...], (((1,), (1,)), ((), ())),
                         preferred_element_type=jnp.float32)
    ys_ref[...] = eo * roww_ref[...]


@jax.jit
def _gmm(te, xs, gup, dnp, roww):
    grid_spec = pltpu.PrefetchScalarGridSpec(
        num_scalar_prefetch=1,
        grid=(G,),
        in_specs=[
            pl.BlockSpec((BM, HID), lambda g, te: (g, 0)),
            pl.BlockSpec((1, 2 * FF, HID), lambda g, te: (te[g], 0, 0)),
            pl.BlockSpec((1, HID, FF), lambda g, te: (te[g], 0, 0)),
            pl.BlockSpec((BM, 1), lambda g, te: (g, 0)),
        ],
        out_specs=pl.BlockSpec((BM, HID), lambda g, te: (g, 0)),
        scratch_shapes=[pltpu.VMEM((2 * FF, HID), jnp.bfloat16),
                        pltpu.VMEM((HID, FF), jnp.bfloat16)],
    )
    return pl.pallas_call(
        _gmm_body,
        grid_spec=grid_spec,
        out_shape=jax.ShapeDtypeStruct((NP, HID), jnp.float32),
        compiler_params=pltpu.CompilerParams(
            dimension_semantics=("arbitrary",)),
    )(te, xs, gup, dnp, roww)


def _shared_body(x_ref, sg_ref, su_ref, sd_ref, gate_ref, out_ref,
                 sgb_ref, sub_ref, sdb_ref):
    @pl.when(pl.program_id(0) == 0)
    def _cast():
        sgb_ref[...] = sg_ref[...].astype(jnp.bfloat16)
        sub_ref[...] = su_ref[...].astype(jnp.bfloat16)
        sdb_ref[...] = sd_ref[...].astype(jnp.bfloat16)

    xb = x_ref[...].astype(jnp.bfloat16)
    gg = lax.dot_general(xb, sgb_ref[...], (((1,), (1,)), ((), ())),
                         preferred_element_type=jnp.float32)
    uu = lax.dot_general(xb, sub_ref[...], (((1,), (1,)), ((), ())),
                         preferred_element_type=jnp.float32)
    h = (gg * (1.0 / (1.0 + jnp.exp(-gg))) * uu).astype(jnp.bfloat16)
    eo = lax.dot_general(h, sdb_ref[...], (((1,), (1,)), ((), ())),
                         preferred_element_type=jnp.float32)
    out_ref[...] = eo * gate_ref[...]


@jax.jit
def _shared(x, sg, su, sd, gate):
    return pl.pallas_call(
        _shared_body,
        grid=(T // TB,),
        in_specs=[
            pl.BlockSpec((TB, HID), lambda g: (g, 0)),
            pl.BlockSpec((FF, HID), lambda g: (0, 0)),
            pl.BlockSpec((FF, HID), lambda g: (0, 0)),
            pl.BlockSpec((HID, FF), lambda g: (0, 0)),
            pl.BlockSpec((TB, 1), lambda g: (g, 0)),
        ],
        out_specs=pl.BlockSpec((TB, HID), lambda g: (g, 0)),
        out_shape=jax.ShapeDtypeStruct((T, HID), jnp.float32),
        scratch_shapes=[pltpu.VMEM((FF, HID), jnp.bfloat16),
                        pltpu.VMEM((FF, HID), jnp.bfloat16),
                        pltpu.VMEM((HID, FF), jnp.bfloat16)],
        compiler_params=pltpu.CompilerParams(
            dimension_semantics=("arbitrary",)),
    )(x, sg, su, sd, gate)


_SW = T // 32


def _comb_body(ys_hbm, sh_hbm, pos_hbm, out_hbm, idx4, abuf, bbuf, cbuf,
               sa, sb):
    c = lax.axis_index("c")
    s = lax.axis_index("s")
    w = c * 16 + s
    tbase = w * _SW
    pltpu.sync_copy(pos_hbm.at[pl.ds(tbase, 32)], idx4.at[0])
    pltpu.sync_copy(pos_hbm.at[pl.ds(tbase + 32, 32)], idx4.at[1])
    pltpu.sync_copy(pos_hbm.at[pl.ds(T + tbase, 32)], idx4.at[2])
    pltpu.sync_copy(pos_hbm.at[pl.ds(T + tbase + 32, 32)], idx4.at[3])
    for h in range(2):
        t0 = tbase + h * 32
        pltpu.async_copy(ys_hbm.at[idx4.at[h]], abuf, sa).wait()
        pltpu.async_copy(ys_hbm.at[idx4.at[2 + h]], bbuf, sb).wait()
        pltpu.sync_copy(sh_hbm.at[pl.ds(t0, 32)], cbuf)

        def row(r, carry):
            for q in range(HID // 16):
                sl = pl.ds(q * 16, 16)
                abuf[r, sl] = abuf[r, sl] + bbuf[r, sl] + cbuf[r, sl]
            return carry
        lax.fori_loop(0, 32, row, 0)
        pltpu.sync_copy(abuf, out_hbm.at[pl.ds(t0, 32)])


@jax.jit
def _combine(ys, sh, pos):
    mesh = plsc.VectorSubcoreMesh(core_axis_name="c", subcore_axis_name="s")
    f = functools.partial(
        pl.kernel,
        out_type=jax.ShapeDtypeStruct((T, HID), jnp.float32),
        mesh=mesh,
        scratch_types=[
            pltpu.VMEM((4, 32), jnp.int32),
            pltpu.VMEM((32, HID), jnp.float32),
            pltpu.VMEM((32, HID), jnp.float32),
            pltpu.VMEM((32, HID), jnp.float32),
            pltpu.SemaphoreType.DMA,
            pltpu.SemaphoreType.DMA,
        ])(_comb_body)
    return f(ys, sh, pos)


def kernel(hidden_states, router_weight, gate_up_proj, down_proj,
           shared_gate_proj, shared_up_proj, shared_down_proj,
           shared_expert_gate_weight):
    B, S, H = hidden_states.shape
    x = hidden_states.reshape(-1, H)
    rwp = jnp.zeros((128, H), jnp.float32)
    rwp = rwp.at[:NE].set(router_weight)
    rwp = rwp.at[NE].set(shared_expert_gate_weight[0])

    pairw, pos, gate, te = _meta(x, rwp)
    pairw1 = pairw.reshape(NPAIR)
    pos1 = pos.reshape(NPAIR)
    te1 = te.reshape(32)[:G]

    xs, roww = _dispatch(x, pos1, pairw1)
    ys = _gmm(te1, xs, gate_up_proj, down_proj, roww.reshape(NP, 1))
    sh = _shared(x, shared_gate_proj, shared_up_proj, shared_down_proj,
                 gate)
    out = _combine(ys, sh, pos1)
    return out.reshape(B, S, H)


# speedup vs baseline: 2.0023x; 1.4547x over previous
import functools

import jax
import jax.numpy as jnp
from jax import lax
from jax.experimental import pallas as pl
from jax.experimental.pallas import tpu as pltpu
from jax.experimental.pallas import tpu_sc as plsc

NE = 8
HID = 1024
FF = 1408
T = 2048
NPAIR = 2 * T
BM = 256
CB = 128
NP = NPAIR + NE * BM
G = NP // BM
TB = 256
NEG = -1e30


def _meta_body(x_ref, rwp_ref, pairw_ref, pos_ref, gate_ref, te_ref,
               oh_ref, rk_ref):
    logits = lax.dot_general(
        x_ref[...], rwp_ref[...], (((1,), (1,)), ((), ())),
        preferred_element_type=jnp.float32)
    lane = lax.broadcasted_iota(jnp.int32, (T, 128), 1)
    l = jnp.where(lane < NE, logits, NEG)
    m0 = jnp.max(l, axis=1, keepdims=True)
    e0 = jnp.min(jnp.where(l >= m0, lane, 9999), axis=1, keepdims=True)
    l2 = jnp.where(lane == e0, NEG, l)
    m1 = jnp.max(l2, axis=1, keepdims=True)
    e1 = jnp.min(jnp.where(l2 >= m1, lane, 9999), axis=1, keepdims=True)
    w0 = 1.0 / (1.0 + jnp.exp(m1 - m0))
    pairw_ref[:T, :] = w0
    pairw_ref[T:, :] = 1.0 - w0
    g = logits[:, NE:NE + 1]
    gate_ref[...] = 1.0 / (1.0 + jnp.exp(-g))

    oh_ref[:T, :] = (lane == e0).astype(jnp.float32)
    oh_ref[T:, :] = (lane == e1).astype(jnp.float32)

    ri = lax.broadcasted_iota(jnp.int32, (CB, CB), 0)
    ci = lax.broadcasted_iota(jnp.int32, (CB, CB), 1)
    stril = (ri > ci).astype(jnp.float32)
    striu = (ri < ci).astype(jnp.float32)
    carry = jnp.zeros((1, 128), jnp.float32)
    for b in range(NPAIR // CB):
        blk = oh_ref[b * CB:(b + 1) * CB, :]
        ex = jnp.dot(stril, blk, preferred_element_type=jnp.float32)
        rk_ref[b * CB:(b + 1) * CB, :] = ex + carry
        carry = carry + jnp.sum(blk, axis=0, keepdims=True)

    counts = carry[0:1, :]
    lr = lax.broadcasted_iota(jnp.int32, (1, 128), 1)
    ci32 = counts.astype(jnp.int32)
    cpad = jnp.where(lr < NE, ((ci32 + BM - 1) // BM) * BM, 0)
    cpadf = cpad.astype(jnp.float32)
    offs = jnp.dot(cpadf, striu, preferred_element_type=jnp.float32)
    oend = jnp.where(lr < NE, offs + cpadf, 1e9)

    for b in range(NPAIR // CB):
        sl = pl.ds(b * CB, CB)
        posb = jnp.sum((rk_ref[sl, :] + offs) * oh_ref[sl, :],
                       axis=1, keepdims=True)
        pos_ref[sl, :] = posb.astype(jnp.int32)

    gcol = lax.broadcasted_iota(jnp.int32, (32, 1), 0) * BM
    te = jnp.sum((gcol.astype(jnp.float32) >= oend).astype(jnp.float32),
                 axis=1, keepdims=True).astype(jnp.int32)
    te_ref[...] = jnp.minimum(te, NE - 1)


@jax.jit
def _meta(x, rwp):
    return pl.pallas_call(
        _meta_body,
        in_specs=[pl.BlockSpec((T, HID), lambda: (0, 0)),
                  pl.BlockSpec((128, HID), lambda: (0, 0))],
        out_specs=[pl.BlockSpec((NPAIR, 1), lambda: (0, 0)),
                   pl.BlockSpec((NPAIR, 1), lambda: (0, 0)),
                   pl.BlockSpec((T, 1), lambda: (0, 0)),
                   pl.BlockSpec((32, 1), lambda: (0, 0))],
        out_shape=[jax.ShapeDtypeStruct((NPAIR, 1), jnp.float32),
                   jax.ShapeDtypeStruct((NPAIR, 1), jnp.int32),
                   jax.ShapeDtypeStruct((T, 1), jnp.float32),
                   jax.ShapeDtypeStruct((32, 1), jnp.int32)],
        scratch_shapes=[pltpu.VMEM((NPAIR, 128), jnp.float32),
                        pltpu.VMEM((NPAIR, 128), jnp.float32)],
    )(x, rwp)


_PW = NPAIR // 16
_RW = NP // 32
_GC = 48
_ZW = NP // 16


def _disp_body(x_hbm, pos_hbm, pairw_hbm, xs_hbm, roww_hbm,
               idxp, idx2, vw, xrows, rbuf, zf, w_sh, sem):
    c = lax.axis_index("c")
    s = lax.axis_index("s")
    w = c * 16 + s

    tbase = w * (T // 32)
    cp = pltpu.async_copy(x_hbm.at[pl.ds(tbase, T // 32)], xrows, sem)
    pltpu.sync_copy(pos_hbm.at[pl.ds(tbase, T // 32)], idxp.at[0])
    pltpu.sync_copy(pos_hbm.at[pl.ds(T + tbase, T // 32)], idxp.at[1])
    cp.wait()
    pltpu.sync_copy(xrows, xs_hbm.at[idxp.at[0]])
    pltpu.sync_copy(xrows, xs_hbm.at[idxp.at[1]])

    for j in range(_ZW // 16):
        zf[pl.ds(j * 16, 16)] = jnp.zeros((16,), jnp.float32)
    pltpu.sync_copy(zf, w_sh.at[pl.ds(s * _ZW, _ZW)])
    plsc.subcore_barrier()
    base = s * _PW
    for j in range(2):
        pltpu.sync_copy(pairw_hbm.at[pl.ds(base + j * 128, 128)], vw.at[j])
        pltpu.sync_copy(pos_hbm.at[pl.ds(base + j * 128, 128)], idx2.at[j])
        pltpu.sync_copy(vw.at[j], w_sh.at[idx2.at[j]], add=True)
    plsc.subcore_barrier()
    rbase = c * (NP // 2) + s * _RW
    pltpu.sync_copy(w_sh.at[pl.ds(rbase, _RW)], rbuf)
    pltpu.sync_copy(rbuf, roww_hbm.at[pl.ds(rbase, _RW)])


@jax.jit
def _dispatch(x, pos, pairw):
    mesh = plsc.VectorSubcoreMesh(core_axis_name="c", subcore_axis_name="s")
    f = functools.partial(
        pl.kernel,
        out_type=(jax.ShapeDtypeStruct((NP, HID), jnp.float32),
                  jax.ShapeDtypeStruct((NP,), jnp.float32)),
        mesh=mesh,
        scratch_types=[
            pltpu.VMEM((2, T // 32), jnp.int32),
            pltpu.VMEM((2, 128), jnp.int32),
            pltpu.VMEM((2, 128), jnp.float32),
            pltpu.VMEM((T // 32, HID), jnp.float32),
            pltpu.VMEM((_RW,), jnp.float32),
            pltpu.VMEM((_ZW,), jnp.float32),
            pltpu.VMEM_SHARED((NP,), jnp.float32),
            pltpu.SemaphoreType.DMA,
        ])(_disp_body)
    return f(x, pos, pairw)


def _gmm_body(te_ref, xs_ref, gu_ref, dn_ref, roww_ref, ys_ref,
              gub_ref, dnb_ref):
    g = pl.program_id(0)
    te = te_ref[g]
    prev = te_ref[jnp.maximum(g - 1, 0)]

    @pl.when((g == 0) | (te != prev))
    def _recast():
        gub_ref[...] = gu_ref[0].astype(jnp.bfloat16)
        dnb_ref[...] = dn_ref[0].astype(jnp.bfloat16)

    xb = xs_ref[...].astype(jnp.bfloat16)
    gu = lax.dot_general(xb, gub_ref[...], (((1,), (1,)), ((), ())),
                         preferred_element_type=jnp.float32)
    h = (gu[:, :FF] * (1.0 / (1.0 + jnp.exp(-gu[:, :FF])))
         * gu[:, FF:]).astype(jnp.bfloat16)
    eo = lax.dot_general(h, dnb_ref[...], (((1,), (1,)), ((), ())),
                         preferred_element_type=jnp.float32)
    ys_ref[...] = eo * roww_ref[...]


@jax.jit
def _gmm(te, xs, gup, dnp, roww):
    grid_spec = pltpu.PrefetchScalarGridSpec(
        num_scalar_prefetch=1,
        grid=(G,),
        in_specs=[
            pl.BlockSpec((BM, HID), lambda g, te: (g, 0)),
            pl.BlockSpec((1, 2 * FF, HID), lambda g, te: (te[g], 0, 0)),
            pl.BlockSpec((1, HID, FF), lambda g, te: (te[g], 0, 0)),
            pl.BlockSpec((BM, 1), lambda g, te: (g, 0)),
        ],
        out_specs=pl.BlockSpec((BM, HID), lambda g, te: (g, 0)),
        scratch_shapes=[pltpu.VMEM((2 * FF, HID), jnp.bfloat16),
                        pltpu.VMEM((HID, FF), jnp.bfloat16)],
    )
    return pl.pallas_call(
        _gmm_body,
        grid_spec=grid_spec,
        out_shape=jax.ShapeDtypeStruct((NP, HID), jnp.float32),
        compiler_params=pltpu.CompilerParams(
            dimension_semantics=("arbitrary",)),
    )(te, xs, gup, dnp, roww)


def _shared_body(x_ref, sg_ref, su_ref, sd_ref, gate_ref, out_ref,
                 sgb_ref, sub_ref, sdb_ref):
    @pl.when(pl.program_id(0) == 0)
    def _cast():
        sgb_ref[...] = sg_ref[...].astype(jnp.bfloat16)
        sub_ref[...] = su_ref[...].astype(jnp.bfloat16)
        sdb_ref[...] = sd_ref[...].astype(jnp.bfloat16)

    xb = x_ref[...].astype(jnp.bfloat16)
    gg = lax.dot_general(xb, sgb_ref[...], (((1,), (1,)), ((), ())),
                         preferred_element_type=jnp.float32)
    uu = lax.dot_general(xb, sub_ref[...], (((1,), (1,)), ((), ())),
                         preferred_element_type=jnp.float32)
    h = (gg * (1.0 / (1.0 + jnp.exp(-gg))) * uu).astype(jnp.bfloat16)
    eo = lax.dot_general(h, sdb_ref[...], (((1,), (1,)), ((), ())),
                         preferred_element_type=jnp.float32)
    out_ref[...] = eo * gate_ref[...]


@jax.jit
def _shared(x, sg, su, sd, gate):
    return pl.pallas_call(
        _shared_body,
        grid=(T // TB,),
        in_specs=[
            pl.BlockSpec((TB, HID), lambda g: (g, 0)),
            pl.BlockSpec((FF, HID), lambda g: (0, 0)),
            pl.BlockSpec((FF, HID), lambda g: (0, 0)),
            pl.BlockSpec((HID, FF), lambda g: (0, 0)),
            pl.BlockSpec((TB, 1), lambda g: (g, 0)),
        ],
        out_specs=pl.BlockSpec((TB, HID), lambda g: (g, 0)),
        out_shape=jax.ShapeDtypeStruct((T, HID), jnp.float32),
        scratch_shapes=[pltpu.VMEM((FF, HID), jnp.bfloat16),
                        pltpu.VMEM((FF, HID), jnp.bfloat16),
                        pltpu.VMEM((HID, FF), jnp.bfloat16)],
        compiler_params=pltpu.CompilerParams(
            dimension_semantics=("arbitrary",)),
    )(x, sg, su, sd, gate)


_SW = T // 32


def _comb_body(ys_hbm, sh_hbm, pos_hbm, out_hbm, idx4, abuf, bbuf, cbuf,
               sa, sb):
    c = lax.axis_index("c")
    s = lax.axis_index("s")
    w = c * 16 + s
    tbase = w * _SW
    pltpu.sync_copy(pos_hbm.at[pl.ds(tbase, 32)], idx4.at[0])
    pltpu.sync_copy(pos_hbm.at[pl.ds(tbase + 32, 32)], idx4.at[1])
    pltpu.sync_copy(pos_hbm.at[pl.ds(T + tbase, 32)], idx4.at[2])
    pltpu.sync_copy(pos_hbm.at[pl.ds(T + tbase + 32, 32)], idx4.at[3])
    for h in range(2):
        t0 = tbase + h * 32
        ca = pltpu.async_copy(ys_hbm.at[idx4.at[h]], abuf, sa)
        cb = pltpu.async_copy(ys_hbm.at[idx4.at[2 + h]], bbuf, sb)
        pltpu.sync_copy(sh_hbm.at[pl.ds(t0, 32)], cbuf)
        ca.wait()
        cb.wait()

        def row(r, carry):
            for q in range(HID // 16):
                sl = pl.ds(q * 16, 16)
                abuf[r, sl] = abuf[r, sl] + bbuf[r, sl] + cbuf[r, sl]
            return carry
        lax.fori_loop(0, 32, row, 0)
        pltpu.sync_copy(abuf, out_hbm.at[pl.ds(t0, 32)])


@jax.jit
def _combine(ys, sh, pos):
    mesh = plsc.VectorSubcoreMesh(core_axis_name="c", subcore_axis_name="s")
    f = functools.partial(
        pl.kernel,
        out_type=jax.ShapeDtypeStruct((T, HID), jnp.float32),
        mesh=mesh,
        scratch_types=[
            pltpu.VMEM((4, 32), jnp.int32),
            pltpu.VMEM((32, HID), jnp.float32),
            pltpu.VMEM((32, HID), jnp.float32),
            pltpu.VMEM((32, HID), jnp.float32),
            pltpu.SemaphoreType.DMA,
            pltpu.SemaphoreType.DMA,
        ])(_comb_body)
    return f(ys, sh, pos)


def kernel(hidden_states, router_weight, gate_up_proj, down_proj,
           shared_gate_proj, shared_up_proj, shared_down_proj,
           shared_expert_gate_weight):
    B, S, H = hidden_states.shape
    x = hidden_states.reshape(-1, H)
    rwp = jnp.zeros((128, H), jnp.float32)
    rwp = rwp.at[:NE].set(router_weight)
    rwp = rwp.at[NE].set(shared_expert_gate_weight[0])

    pairw, pos, gate, te = _meta(x, rwp)
    pairw1 = pairw.reshape(NPAIR)
    pos1 = pos.reshape(NPAIR)
    te1 = te.reshape(32)[:G]

    xs, roww = _dispatch(x, pos1, pairw1)
    ys = _gmm(te1, xs, gate_up_proj, down_proj, roww.reshape(NP, 1))
    sh = _shared(x, shared_gate_proj, shared_up_proj, shared_down_proj,
                 gate)
    out = _combine(ys, sh, pos1)
    return out.reshape(B, S, H)
